# baseline (speedup 1.0000x reference)
import functools

import jax
import jax.numpy as jnp
from jax import lax
from jax.experimental import pallas as pl
from jax.experimental.pallas import tpu as pltpu
from jax.experimental.pallas import tpu_sc as plsc

NC = 1

NS = 16
L = 16
NW = NC * NS

B = 16384
TAB = 1000
BPW = B // NW
VECS = BPW // L


def _softplus_floor(s):
    e = jnp.exp(-jnp.abs(s))
    u = e / (e + 2.0)
    t = u * u
    p = 1.0 / 11.0
    p = p * t + 1.0 / 9.0
    p = p * t + 1.0 / 7.0
    p = p * t + 1.0 / 5.0
    p = p * t + 1.0 / 3.0
    p = p * t + 1.0
    log1p_e = 2.0 * u * p
    sp = jnp.maximum(s, 0.0) + log1p_e
    return jnp.maximum(sp, 0.001)


@functools.partial(
    pl.kernel,
    out_type=(
        jax.ShapeDtypeStruct((B,), jnp.float32),
        jax.ShapeDtypeStruct((B,), jnp.float32),
    ),
    mesh=plsc.VectorSubcoreMesh(
        core_axis_name="c", subcore_axis_name="s", num_cores=NC, num_subcores=NS
    ),
    compiler_params=pltpu.CompilerParams(needs_layout_passes=False),
    scratch_types=[
        pltpu.VMEM((TAB,), jnp.float32),
        pltpu.VMEM((TAB,), jnp.float32),
        pltpu.VMEM((BPW,), jnp.int32),
        pltpu.VMEM((BPW,), jnp.float32),
        pltpu.VMEM((BPW,), jnp.float32),
        pltpu.SemaphoreType.DMA,
    ],
)
def _gather_softplus(loc_hbm, scale_hbm, idx_hbm, out_loc, out_scale,
                     loc_tab, scale_tab, idx_v, oloc_v, oscale_v, sem):
    wid = lax.axis_index("s") * NC + lax.axis_index("c")
    base = wid * BPW
    c1 = pltpu.async_copy(idx_hbm.at[pl.ds(base, BPW)], idx_v, sem)
    c2 = pltpu.async_copy(loc_hbm, loc_tab, sem)
    c3 = pltpu.async_copy(scale_hbm, scale_tab, sem)
    c1.wait()
    c2.wait()
    c3.wait()

    for j in range(VECS):
        off = j * L
        idx = idx_v[pl.ds(off, L)]
        lv = plsc.load_gather(loc_tab, [idx])
        sv = plsc.load_gather(scale_tab, [idx])
        oloc_v[pl.ds(off, L)] = lv
        oscale_v[pl.ds(off, L)] = _softplus_floor(sv)

    c4 = pltpu.async_copy(oloc_v, out_loc.at[pl.ds(base, BPW)], sem)
    c5 = pltpu.async_copy(oscale_v, out_scale.at[pl.ds(base, BPW)], sem)
    c4.wait()
    c5.wait()


def kernel(x, diag_loc, diag_scale):
    loc, scale = _gather_softplus(
        diag_loc.reshape(-1), diag_scale.reshape(-1), x.astype(jnp.int32))
    return loc.reshape(-1, 1), scale.reshape(-1, 1)


# speedup vs baseline: 1.1497x; 1.1497x over previous
import functools

import jax
import jax.numpy as jnp
from jax import lax
from jax.experimental import pallas as pl
from jax.experimental.pallas import tpu as pltpu
from jax.experimental.pallas import tpu_sc as plsc

NC = 2
NS = 16
L = 16
NW = NC * NS

B = 16384
TAB = 1024
BPW = B // NW
VECS = BPW // L
TPW = TAB // NS
TVECS = TPW // L


def _softplus_floor(s):
    e = jnp.exp(-jnp.abs(s))
    u = e / (e + 2.0)
    t = u * u
    p = 1.0 / 11.0
    p = p * t + 1.0 / 9.0
    p = p * t + 1.0 / 7.0
    p = p * t + 1.0 / 5.0
    p = p * t + 1.0 / 3.0
    p = p * t + 1.0
    log1p_e = 2.0 * u * p
    sp = jnp.maximum(s, 0.0) + log1p_e
    return jnp.maximum(sp, 0.001)


@functools.partial(
    pl.kernel,
    out_type=(
        jax.ShapeDtypeStruct((B,), jnp.float32),
        jax.ShapeDtypeStruct((B,), jnp.float32),
    ),
    mesh=plsc.VectorSubcoreMesh(
        core_axis_name="c", subcore_axis_name="s", num_cores=NC, num_subcores=NS
    ),
    compiler_params=pltpu.CompilerParams(needs_layout_passes=False),
    scratch_types=[
        pltpu.VMEM((TAB,), jnp.float32),
        pltpu.VMEM((TAB,), jnp.float32),
        pltpu.VMEM((TPW,), jnp.float32),
        pltpu.VMEM((BPW,), jnp.int32),
        pltpu.VMEM((BPW,), jnp.float32),
        pltpu.VMEM((BPW,), jnp.float32),
        pltpu.VMEM_SHARED((TAB,), jnp.float32),
        pltpu.SemaphoreType.DMA,
        pltpu.SemaphoreType.DMA,
    ],
)
def _gather_softplus(loc_hbm, scale_hbm, idx_hbm, out_loc, out_scale,
                     loc_tab, scale_tab, slice_v, idx_v, oloc_v, oscale_v,
                     scale_sp, sem, sem_scale):
    sid = lax.axis_index("s")
    wid = sid * NC + lax.axis_index("c")
    base = wid * BPW
    c1 = pltpu.async_copy(idx_hbm.at[pl.ds(base, BPW)], idx_v, sem)
    c2 = pltpu.async_copy(loc_hbm, loc_tab, sem)
    c3 = pltpu.async_copy(scale_hbm, scale_tab, sem_scale)
    c3.wait()

    toff = sid * TPW
    for v in range(TVECS):
        sv = scale_tab[pl.ds(toff + v * L, L)]
        slice_v[pl.ds(v * L, L)] = _softplus_floor(sv)
    pltpu.sync_copy(slice_v, scale_sp.at[pl.ds(toff, TPW)])
    plsc.subcore_barrier()
    c4 = pltpu.async_copy(scale_sp, scale_tab, sem)
    c1.wait()
    c2.wait()
    c4.wait()

    for j in range(VECS):
        off = j * L
        idx = idx_v[pl.ds(off, L)]
        oloc_v[pl.ds(off, L)] = plsc.load_gather(loc_tab, [idx])
        oscale_v[pl.ds(off, L)] = plsc.load_gather(scale_tab, [idx])

    c5 = pltpu.async_copy(oloc_v, out_loc.at[pl.ds(base, BPW)], sem)
    c6 = pltpu.async_copy(oscale_v, out_scale.at[pl.ds(base, BPW)], sem)
    c5.wait()
    c6.wait()


def kernel(x, diag_loc, diag_scale):
    loc_t = jnp.pad(diag_loc.reshape(-1), (0, TAB - diag_loc.shape[0]))
    scale_t = jnp.pad(diag_scale.reshape(-1), (0, TAB - diag_scale.shape[0]))
    loc, scale = _gather_softplus(loc_t, scale_t, x.astype(jnp.int32))
    return loc.reshape(-1, 1), scale.reshape(-1, 1)
